# baseline (speedup 1.0000x reference)
import functools

import jax
import jax.numpy as jnp
from jax import lax
from jax.experimental import pallas as pl
from jax.experimental.pallas import tpu as pltpu, tpu_sc as plsc

N_PSI = 192
N_CONES = 12288
N_PART = 1048576
NUM_CORES = 2
NUM_SUBCORES = 16
NW = NUM_CORES * NUM_SUBCORES
B_PER_W = N_PART // NW
L = 16
CH = 4096
NCH = B_PER_W // CH

_mesh = plsc.VectorSubcoreMesh(core_axis_name="c", subcore_axis_name="s")


def _rne_bf16_hi(u):
    odd = lax.shift_right_logical(u, 16) & jnp.int32(1)
    return u + jnp.int32(0x7FFF) + odd


@functools.partial(
    pl.kernel,
    mesh=_mesh,
    out_type=jax.ShapeDtypeStruct((N_PART,), jnp.float32),
    scratch_types=[
        pltpu.VMEM((CH,), jnp.int32),
        pltpu.VMEM((CH,), jnp.int32),
        pltpu.VMEM((CH,), jnp.float32),
        pltpu.VMEM((CH,), jnp.float32),
        pltpu.VMEM((CH,), jnp.float32),
        pltpu.VMEM((N_CONES,), jnp.float32),
        pltpu.VMEM((N_CONES,), jnp.float32),
        pltpu.VMEM((N_CONES,), jnp.int32),
        pltpu.SemaphoreType.DMA,
        pltpu.SemaphoreType.DMA,
        pltpu.SemaphoreType.DMA,
        pltpu.SemaphoreType.DMA,
    ],
    compiler_params=pltpu.CompilerParams(needs_layout_passes=False),
)
def _normalize(idx_hbm, scores_hbm, med_hbm, mad_hbm, out_hbm,
               idx_v0, idx_v1, sc_v0, sc_v1, sc_v2, med_v, mad_v, pk_v,
               sem_t, sem_i, sem_s, sem_o):
    idx_bufs = (idx_v0, idx_v1)
    sc_bufs = (sc_v0, sc_v1, sc_v2)
    wid = lax.axis_index("s") * NUM_CORES + lax.axis_index("c")
    base = wid * B_PER_W
    cp_med = pltpu.async_copy(med_hbm, med_v, sem_t)
    cp_mad = pltpu.async_copy(mad_hbm, mad_v, sem_t)

    def issue_in(c):
        o = base + c * CH
        return (
            pltpu.async_copy(idx_hbm.at[pl.ds(o, CH)], idx_bufs[c % 2], sem_i),
            pltpu.async_copy(scores_hbm.at[pl.ds(o, CH)], sc_bufs[c % 3], sem_s),
        )

    in_cp = [issue_in(0), issue_in(1)]
    cp_med.wait()
    cp_mad.wait()

    @plsc.parallel_loop(0, N_CONES, step=L, unroll=8)
    def _prep(off):
        r = 1.0 / mad_v[pl.ds(off, L)]
        m = med_v[pl.ds(off, L)] * r
        rr = _rne_bf16_hi(plsc.bitcast(r, jnp.int32))
        rm = _rne_bf16_hi(plsc.bitcast(m, jnp.int32))
        pk_v[pl.ds(off, L)] = (rm & jnp.int32(-65536)) | lax.shift_right_logical(rr, 16)

    out_cp = []
    for c in range(NCH):
        cp_i, cp_s = in_cp[c]
        cp_i.wait()
        cp_s.wait()
        idx_ref = idx_bufs[c % 2]
        sc_ref = sc_bufs[c % 3]

        @plsc.parallel_loop(0, CH, step=L, unroll=8)
        def _step(off, idx_ref=idx_ref, sc_ref=sc_ref):
            so3 = idx_ref[pl.ds(off, L)]
            q6 = lax.shift_right_logical(so3, 6)
            cone = lax.shift_right_logical(q6 * jnp.int32(43691), 17)
            w = plsc.load_gather(pk_v, [cone])
            rmad = plsc.bitcast(lax.shift_left(w, 16), jnp.float32)
            medr = plsc.bitcast(w & jnp.int32(-65536), jnp.float32)
            sc_ref[pl.ds(off, L)] = sc_ref[pl.ds(off, L)] * rmad - medr

        out_cp.append(
            pltpu.async_copy(sc_ref, out_hbm.at[pl.ds(base + c * CH, CH)], sem_o))
        if c + 2 < NCH:
            if c >= 1:
                out_cp[c - 1].wait()
            in_cp.append(issue_in(c + 2))
    out_cp[NCH - 3].wait()
    out_cp[NCH - 2].wait()
    out_cp[NCH - 1].wait()


def kernel(so3_indices, scores, medians, mads):
    return _normalize(so3_indices, scores, medians, mads)


# speedup vs baseline: 1.0017x; 1.0017x over previous
import functools

import jax
import jax.numpy as jnp
from jax import lax
from jax.experimental import pallas as pl
from jax.experimental.pallas import tpu as pltpu, tpu_sc as plsc

N_PSI = 192
N_CONES = 12288
N_PART = 1048576
NUM_CORES = 2
NUM_SUBCORES = 16
NW = NUM_CORES * NUM_SUBCORES
B_PER_W = N_PART // NW
L = 16
CH = 4096
NCH = B_PER_W // CH

_mesh = plsc.VectorSubcoreMesh(core_axis_name="c", subcore_axis_name="s")


def _rne_bf16_hi(u):
    odd = lax.shift_right_logical(u, 16) & jnp.int32(1)
    return u + jnp.int32(0x7FFF) + odd


@functools.partial(
    pl.kernel,
    mesh=_mesh,
    out_type=jax.ShapeDtypeStruct((N_PART,), jnp.float32),
    scratch_types=[
        pltpu.VMEM((CH,), jnp.int32),
        pltpu.VMEM((CH,), jnp.int32),
        pltpu.VMEM((CH,), jnp.float32),
        pltpu.VMEM((CH,), jnp.float32),
        pltpu.VMEM((CH,), jnp.float32),
        pltpu.VMEM((N_CONES,), jnp.float32),
        pltpu.VMEM((N_CONES,), jnp.float32),
        pltpu.VMEM((N_CONES,), jnp.int32),
        pltpu.SemaphoreType.DMA,
        pltpu.SemaphoreType.DMA,
        pltpu.SemaphoreType.DMA,
        pltpu.SemaphoreType.DMA,
    ],
    compiler_params=pltpu.CompilerParams(needs_layout_passes=False),
)
def _normalize(idx_hbm, scores_hbm, med_hbm, mad_hbm, out_hbm,
               idx_v0, idx_v1, sc_v0, sc_v1, sc_v2, med_v, mad_v, pk_v,
               sem_t, sem_i, sem_s, sem_o):
    idx_bufs = (idx_v0, idx_v1)
    sc_bufs = (sc_v0, sc_v1, sc_v2)
    wid = lax.axis_index("s") * NUM_CORES + lax.axis_index("c")
    base = wid * B_PER_W
    cp_med = pltpu.async_copy(med_hbm, med_v, sem_t)
    cp_mad = pltpu.async_copy(mad_hbm, mad_v, sem_t)

    def issue_in(c):
        o = base + c * CH
        return (
            pltpu.async_copy(idx_hbm.at[pl.ds(o, CH)], idx_bufs[c % 2], sem_i),
            pltpu.async_copy(scores_hbm.at[pl.ds(o, CH)], sc_bufs[c % 3], sem_s),
        )

    in_cp = [issue_in(0), issue_in(1)]
    with jax.named_scope("tbl_wait"):
        cp_med.wait()
        cp_mad.wait()

    with jax.named_scope("prep"):
        _run_prep(med_v, mad_v, pk_v)

    out_cp = []
    for c in range(NCH):
        cp_i, cp_s = in_cp[c]
        with jax.named_scope("in_wait"):
            cp_i.wait()
            cp_s.wait()
        idx_ref = idx_bufs[c % 2]
        sc_ref = sc_bufs[c % 3]
        with jax.named_scope("hot"):
            _run_chunk(idx_ref, sc_ref, pk_v)
        out_cp.append(
            pltpu.async_copy(sc_ref, out_hbm.at[pl.ds(base + c * CH, CH)], sem_o))
        if c + 2 < NCH:
            if c >= 1:
                with jax.named_scope("out_wait"):
                    out_cp[c - 1].wait()
            in_cp.append(issue_in(c + 2))
    with jax.named_scope("drain"):
        out_cp[NCH - 3].wait()
        out_cp[NCH - 2].wait()
        out_cp[NCH - 1].wait()


def _run_prep(med_v, mad_v, pk_v):
    @plsc.parallel_loop(0, N_CONES, step=L, unroll=8)
    def _prep(off):
        r = 1.0 / mad_v[pl.ds(off, L)]
        m = med_v[pl.ds(off, L)] * r
        rr = _rne_bf16_hi(plsc.bitcast(r, jnp.int32))
        rm = _rne_bf16_hi(plsc.bitcast(m, jnp.int32))
        pk_v[pl.ds(off, L)] = (rm & jnp.int32(-65536)) | lax.shift_right_logical(rr, 16)


def _run_chunk(idx_ref, sc_ref, pk_v):
    @plsc.parallel_loop(0, CH, step=L, unroll=8)
    def _step(off):
        so3 = idx_ref[pl.ds(off, L)]
        q6 = lax.shift_right_logical(so3, 6)
        cone = lax.shift_right_logical(q6 * jnp.int32(43691), 17)
        w = plsc.load_gather(pk_v, [cone])
        rmad = plsc.bitcast(lax.shift_left(w, 16), jnp.float32)
        medr = plsc.bitcast(w & jnp.int32(-65536), jnp.float32)
        sc_ref[pl.ds(off, L)] = sc_ref[pl.ds(off, L)] * rmad - medr


def kernel(so3_indices, scores, medians, mads):
    return _normalize(so3_indices, scores, medians, mads)


# speedup vs baseline: 1.1790x; 1.1770x over previous
import functools

import jax
import jax.numpy as jnp
from jax import lax
from jax.experimental import pallas as pl
from jax.experimental.pallas import tpu as pltpu, tpu_sc as plsc

N_PSI = 192
N_CONES = 12288
N_PART = 1048576
NUM_CORES = 2
NUM_SUBCORES = 16
NW = NUM_CORES * NUM_SUBCORES
B_PER_W = N_PART // NW
L = 16
CH = 4096
NCH = B_PER_W // CH
TS = N_CONES // NUM_SUBCORES

_mesh = plsc.VectorSubcoreMesh(core_axis_name="c", subcore_axis_name="s")


def _rne_bf16_hi(u):
    odd = lax.shift_right_logical(u, 16) & jnp.int32(1)
    return u + jnp.int32(0x7FFF) + odd


@functools.partial(
    pl.kernel,
    mesh=_mesh,
    out_type=jax.ShapeDtypeStruct((N_PART,), jnp.float32),
    scratch_types=[
        pltpu.VMEM((CH,), jnp.int32),
        pltpu.VMEM((CH,), jnp.int32),
        pltpu.VMEM((CH,), jnp.float32),
        pltpu.VMEM((CH,), jnp.float32),
        pltpu.VMEM((CH,), jnp.float32),
        pltpu.VMEM((TS,), jnp.float32),
        pltpu.VMEM((TS,), jnp.float32),
        pltpu.VMEM((TS,), jnp.int32),
        pltpu.VMEM((N_CONES,), jnp.int32),
        pltpu.VMEM_SHARED((N_CONES,), jnp.int32),
        pltpu.SemaphoreType.DMA,
        pltpu.SemaphoreType.DMA,
        pltpu.SemaphoreType.DMA,
        pltpu.SemaphoreType.DMA,
    ],
    compiler_params=pltpu.CompilerParams(needs_layout_passes=False),
)
def _normalize(idx_hbm, scores_hbm, med_hbm, mad_hbm, out_hbm,
               idx_v0, idx_v1, sc_v0, sc_v1, sc_v2,
               med_v, mad_v, pk_s, pk_v, pk_sh,
               sem_t, sem_i, sem_s, sem_o):
    idx_bufs = (idx_v0, idx_v1)
    sc_bufs = (sc_v0, sc_v1, sc_v2)
    sid = lax.axis_index("s")
    wid = sid * NUM_CORES + lax.axis_index("c")
    base = wid * B_PER_W
    tbase = sid * TS
    cp_med = pltpu.async_copy(med_hbm.at[pl.ds(tbase, TS)], med_v, sem_t)
    cp_mad = pltpu.async_copy(mad_hbm.at[pl.ds(tbase, TS)], mad_v, sem_t)

    def issue_in(c):
        o = base + c * CH
        return (
            pltpu.async_copy(idx_hbm.at[pl.ds(o, CH)], idx_bufs[c % 2], sem_i),
            pltpu.async_copy(scores_hbm.at[pl.ds(o, CH)], sc_bufs[c % 3], sem_s),
        )

    in_cp = [issue_in(0), issue_in(1)]
    with jax.named_scope("tbl_wait"):
        cp_med.wait()
        cp_mad.wait()

    with jax.named_scope("prep"):
        _run_prep(med_v, mad_v, pk_s)
        pltpu.sync_copy(pk_s, pk_sh.at[pl.ds(tbase, TS)])
        plsc.subcore_barrier()
        pltpu.sync_copy(pk_sh, pk_v)

    out_cp = []
    for c in range(NCH):
        cp_i, cp_s = in_cp[c]
        with jax.named_scope("in_wait"):
            cp_i.wait()
            cp_s.wait()
        idx_ref = idx_bufs[c % 2]
        sc_ref = sc_bufs[c % 3]
        with jax.named_scope("hot"):
            _run_chunk(idx_ref, sc_ref, pk_v)
        out_cp.append(
            pltpu.async_copy(sc_ref, out_hbm.at[pl.ds(base + c * CH, CH)], sem_o))
        if c + 2 < NCH:
            if c >= 1:
                with jax.named_scope("out_wait"):
                    out_cp[c - 1].wait()
            in_cp.append(issue_in(c + 2))
    with jax.named_scope("drain"):
        out_cp[NCH - 3].wait()
        out_cp[NCH - 2].wait()
        out_cp[NCH - 1].wait()


def _run_prep(med_v, mad_v, pk_s):
    @plsc.parallel_loop(0, TS, step=L, unroll=8)
    def _prep(off):
        r = 1.0 / mad_v[pl.ds(off, L)]
        m = med_v[pl.ds(off, L)] * r
        rr = _rne_bf16_hi(plsc.bitcast(r, jnp.int32))
        rm = _rne_bf16_hi(plsc.bitcast(m, jnp.int32))
        pk_s[pl.ds(off, L)] = (rm & jnp.int32(-65536)) | lax.shift_right_logical(rr, 16)


def _run_chunk(idx_ref, sc_ref, pk_v):
    @plsc.parallel_loop(0, CH, step=L, unroll=8)
    def _step(off):
        so3 = idx_ref[pl.ds(off, L)]
        q6 = lax.shift_right_logical(so3, 6)
        cone = lax.shift_right_logical(q6 * jnp.int32(43691), 17)
        w = plsc.load_gather(pk_v, [cone])
        rmad = plsc.bitcast(lax.shift_left(w, 16), jnp.float32)
        medr = plsc.bitcast(w & jnp.int32(-65536), jnp.float32)
        sc_ref[pl.ds(off, L)] = sc_ref[pl.ds(off, L)] * rmad - medr


def kernel(so3_indices, scores, medians, mads):
    return _normalize(so3_indices, scores, medians, mads)


# speedup vs baseline: 1.2462x; 1.0570x over previous
import functools

import jax
import jax.numpy as jnp
from jax import lax
from jax.experimental import pallas as pl
from jax.experimental.pallas import tpu as pltpu, tpu_sc as plsc

N_PSI = 192
N_CONES = 12288
N_PART = 1048576
NUM_CORES = 2
NUM_SUBCORES = 16
NW = NUM_CORES * NUM_SUBCORES
B_PER_W = N_PART // NW
L = 16
CH = 4096
NCH = B_PER_W // CH
TS = N_CONES // NUM_SUBCORES
NIC = 3
NSC = 4
PF = 3

_mesh = plsc.VectorSubcoreMesh(core_axis_name="c", subcore_axis_name="s")


def _rne_bf16_hi(u):
    odd = lax.shift_right_logical(u, 16) & jnp.int32(1)
    return u + jnp.int32(0x7FFF) + odd


@functools.partial(
    pl.kernel,
    mesh=_mesh,
    out_type=jax.ShapeDtypeStruct((N_PART,), jnp.float32),
    scratch_types=[
        pltpu.VMEM((CH,), jnp.int32),
        pltpu.VMEM((CH,), jnp.int32),
        pltpu.VMEM((CH,), jnp.int32),
        pltpu.VMEM((CH,), jnp.float32),
        pltpu.VMEM((CH,), jnp.float32),
        pltpu.VMEM((CH,), jnp.float32),
        pltpu.VMEM((CH,), jnp.float32),
        pltpu.VMEM((TS,), jnp.float32),
        pltpu.VMEM((TS,), jnp.float32),
        pltpu.VMEM((TS,), jnp.int32),
        pltpu.VMEM((N_CONES,), jnp.int32),
        pltpu.VMEM_SHARED((N_CONES,), jnp.int32),
        pltpu.SemaphoreType.DMA,
        pltpu.SemaphoreType.DMA,
        pltpu.SemaphoreType.DMA,
        pltpu.SemaphoreType.DMA,
    ],
    compiler_params=pltpu.CompilerParams(needs_layout_passes=False),
)
def _normalize(idx_hbm, scores_hbm, med_hbm, mad_hbm, out_hbm,
               idx_v0, idx_v1, idx_v2, sc_v0, sc_v1, sc_v2, sc_v3,
               med_v, mad_v, pk_s, pk_v, pk_sh,
               sem_t, sem_i, sem_s, sem_o):
    idx_bufs = (idx_v0, idx_v1, idx_v2)
    sc_bufs = (sc_v0, sc_v1, sc_v2, sc_v3)
    sid = lax.axis_index("s")
    wid = sid * NUM_CORES + lax.axis_index("c")
    base = wid * B_PER_W
    tbase = sid * TS
    cp_med = pltpu.async_copy(med_hbm.at[pl.ds(tbase, TS)], med_v, sem_t)
    cp_mad = pltpu.async_copy(mad_hbm.at[pl.ds(tbase, TS)], mad_v, sem_t)

    def issue_in(c):
        o = base + c * CH
        return (
            pltpu.async_copy(idx_hbm.at[pl.ds(o, CH)], idx_bufs[c % NIC], sem_i),
            pltpu.async_copy(scores_hbm.at[pl.ds(o, CH)], sc_bufs[c % NSC], sem_s),
        )

    in_cp = [issue_in(c) for c in range(PF)]
    cp_med.wait()
    cp_mad.wait()

    _run_prep(med_v, mad_v, pk_s)
    pltpu.sync_copy(pk_s, pk_sh.at[pl.ds(tbase, TS)])
    plsc.subcore_barrier()
    pltpu.sync_copy(pk_sh, pk_v)

    out_cp = []
    for c in range(NCH):
        cp_i, cp_s = in_cp[c]
        cp_i.wait()
        cp_s.wait()
        idx_ref = idx_bufs[c % NIC]
        sc_ref = sc_bufs[c % NSC]
        _run_chunk(idx_ref, sc_ref, pk_v)
        out_cp.append(
            pltpu.async_copy(sc_ref, out_hbm.at[pl.ds(base + c * CH, CH)], sem_o))
        if c + PF < NCH:
            if c + PF - NSC >= 0:
                out_cp[c + PF - NSC].wait()
            in_cp.append(issue_in(c + PF))
    first_unwaited = max(0, (NCH - PF - 1) + PF - NSC + 1)
    for c in range(first_unwaited, NCH):
        out_cp[c].wait()


def _run_prep(med_v, mad_v, pk_s):
    @plsc.parallel_loop(0, TS, step=L, unroll=8)
    def _prep(off):
        r = 1.0 / mad_v[pl.ds(off, L)]
        m = med_v[pl.ds(off, L)] * r
        rr = _rne_bf16_hi(plsc.bitcast(r, jnp.int32))
        rm = _rne_bf16_hi(plsc.bitcast(m, jnp.int32))
        pk_s[pl.ds(off, L)] = (rm & jnp.int32(-65536)) | lax.shift_right_logical(rr, 16)


def _run_chunk(idx_ref, sc_ref, pk_v):
    @plsc.parallel_loop(0, CH, step=L, unroll=8)
    def _step(off):
        so3 = idx_ref[pl.ds(off, L)]
        q6 = lax.shift_right_logical(so3, 6)
        cone = lax.shift_right_logical(q6 * jnp.int32(43691), 17)
        w = plsc.load_gather(pk_v, [cone])
        rmad = plsc.bitcast(lax.shift_left(w, 16), jnp.float32)
        medr = plsc.bitcast(w & jnp.int32(-65536), jnp.float32)
        sc_ref[pl.ds(off, L)] = sc_ref[pl.ds(off, L)] * rmad - medr


def kernel(so3_indices, scores, medians, mads):
    return _normalize(so3_indices, scores, medians, mads)
